# baseline (speedup 1.0000x reference)
import functools

import jax
import jax.numpy as jnp
from jax import lax
from jax.experimental import pallas as pl
from jax.experimental.pallas import tpu as pltpu
from jax.experimental.pallas import tpu_sc as plsc

N_NODES = 50000
N_EDGES = 1600000
B = 1024
D = 128
T_FEAT = 100
E_FEAT = 16
K = 20

NEG = -3.0e38
NSC = 392
NMACRO = 48
CAP = 96


def _srt(v, i):
    return plsc.sort_key_val(v, i, descending=True)


def _rev(x):
    return lax.rev(x, (0,))


def _merge16_into32(T0v, T0i, T1v, T1i, xv, xi):
    xrv, xri = _rev(xv), _rev(xi)
    s1 = xrv > T0v
    hv = jnp.where(s1, xrv, T0v)
    hi = jnp.where(s1, xri, T0i)
    lv = jnp.where(s1, T0v, xrv)
    li = jnp.where(s1, T0i, xri)
    nT0v, nT0i = _srt(hv, hi)
    lvs, lis = _srt(lv, li)
    lrv, lri = _rev(lvs), _rev(lis)
    s2 = lrv > T1v
    h2v = jnp.where(s2, lrv, T1v)
    h2i = jnp.where(s2, lri, T1i)
    nT1v, nT1i = _srt(h2v, h2i)
    return nT0v, nT0i, nT1v, nT1i


def _m32(a, b, ii):
    rb = _rev(b)
    hi = jnp.maximum(a, rb)
    lo = jnp.minimum(a, rb)
    hi, _ = _srt(hi, ii)
    lo, _ = _srt(lo, ii)
    return hi, lo


def _m32keep(a0, a1, b0, b1, ii):
    c0 = jnp.maximum(a0, _rev(b1))
    c1 = jnp.maximum(a1, _rev(b0))
    t0 = jnp.maximum(c0, c1)
    t1 = jnp.minimum(c0, c1)
    t0, _ = _srt(t0, ii)
    t1, _ = _srt(t1, ii)
    return t0, t1


def _topk_call(tflat):
    info = plsc.get_sparse_core_info()
    NW = info.num_cores * info.num_subcores
    RPW = B // NW
    mesh = plsc.VectorSubcoreMesh(core_axis_name="c", subcore_axis_name="s")

    @functools.partial(
        pl.kernel,
        out_type=(jax.ShapeDtypeStruct((B, 32), jnp.int32),
                  jax.ShapeDtypeStruct((B, 32), jnp.float32)),
        mesh=mesh,
        scratch_types=[
            pltpu.VMEM((N_NODES,), jnp.float32),
            pltpu.VMEM((N_NODES,), jnp.float32),
            pltpu.VMEM((NSC * 16,), jnp.float32),
            pltpu.VMEM(((NMACRO + 1) * 16,), jnp.float32),
            pltpu.VMEM((CAP * 16,), jnp.float32),
            pltpu.VMEM((CAP * 16,), jnp.int32),
            pltpu.VMEM((16,), jnp.int32),
            pltpu.VMEM((16,), jnp.float32),
            pltpu.VMEM((16,), jnp.int32),
            pltpu.VMEM((16,), jnp.float32),
            pltpu.SMEM((8,), jnp.int32),
            pltpu.SemaphoreType.DMA,
            pltpu.SemaphoreType.DMA,
        ],
        compiler_params=pltpu.CompilerParams(needs_layout_passes=False),
    )
    def topk_kernel(tppr_hbm, idx_out, w_out, bufA, bufB, Mref, GMref,
                    cvref, ciref,
                    st_i0, st_w0, st_i1, st_w1, jref, semA, semB):
        wid = lax.axis_index("s") * info.num_cores + lax.axis_index("c")
        r0 = wid * RPW
        ii = lax.iota(jnp.int32, 16)

        def src(row):
            return tppr_hbm.at[row]

        def anyge(x, thr):
            cnt = plsc.all_reduce_population_count(x >= thr)
            return lax.squeeze(lax.slice(cnt, (0,), (1,)), (0,)) > 0

        def process(buf, row):
            def macro(i, accs):
                out = []
                ms = []
                for a in range(8):
                    s = i * 8 + a
                    off = s * 128
                    l = [buf[pl.ds(off + j * 16, 16)] for j in range(8)]
                    m = jnp.maximum(
                        jnp.maximum(jnp.maximum(l[0], l[1]),
                                    jnp.maximum(l[2], l[3])),
                        jnp.maximum(jnp.maximum(l[4], l[5]),
                                    jnp.maximum(l[6], l[7])))
                    Mref[pl.ds(s * 16, 16)] = m
                    ms.append(m)
                    out.append(jnp.maximum(accs[a], m))
                gm = jnp.maximum(
                    jnp.maximum(jnp.maximum(ms[0], ms[1]),
                                jnp.maximum(ms[2], ms[3])),
                    jnp.maximum(jnp.maximum(ms[4], ms[5]),
                                jnp.maximum(ms[6], ms[7])))
                GMref[pl.ds(i * 16, 16)] = gm
                return tuple(out)

            neg16 = jnp.full((16,), NEG, jnp.float32)
            accs = lax.fori_loop(0, NMACRO, macro, (neg16,) * 8)
            accs = list(accs)
            ms = []
            for a in range(7):
                s = 384 + a
                off = s * 128
                nl = 8 if a < 6 else 5
                l = [buf[pl.ds(off + j * 16, 16)] for j in range(nl)]
                m = l[0]
                for j in range(1, nl):
                    m = jnp.maximum(m, l[j])
                Mref[pl.ds(s * 16, 16)] = m
                ms.append(m)
                accs[a] = jnp.maximum(accs[a], m)
            Mref[pl.ds(391 * 16, 16)] = neg16
            gm = ms[0]
            for a in range(1, 7):
                gm = jnp.maximum(gm, ms[a])
            GMref[pl.ds(NMACRO * 16, 16)] = gm

            sa = [_srt(accs[a], ii)[0] for a in range(8)]
            p0 = _m32(sa[0], sa[1], ii)
            p1 = _m32(sa[2], sa[3], ii)
            p2 = _m32(sa[4], sa[5], ii)
            p3 = _m32(sa[6], sa[7], ii)
            q0 = _m32keep(*p0, *p1, ii)
            q1 = _m32keep(*p2, *p3, ii)
            r_ = _m32keep(*q0, *q1, ii)
            thr = jnp.max(jnp.where(ii == 3, r_[1], NEG))

            jref[0] = 0

            def scan_chunk(off):
                x = buf[pl.ds(off, 16)]

                @pl.when(anyge(x, thr))
                def _():
                    jc = jnp.minimum(jref[0], CAP - 1)
                    cvref[pl.ds(jc * 16, 16)] = x
                    ciref[pl.ds(jc * 16, 16)] = ii + off
                    jref[0] = jc + 1

            def scan_grp(g, carry):
                gmv = GMref[pl.ds(g * 16, 16)]

                @pl.when(anyge(gmv, thr))
                def _():
                    for a in range(8):
                        s = g * 8 + a
                        m = Mref[pl.ds(s * 16, 16)]

                        @pl.when(anyge(m, thr))
                        def _():
                            for j in range(8):
                                scan_chunk(s * 128 + j * 16)
                return carry

            lax.fori_loop(0, NMACRO, scan_grp, 0)
            for a in range(7):
                s = 384 + a
                nl = 8 if a < 6 else 5

                def _tail(s=s, nl=nl, m=ms[a]):
                    @pl.when(anyge(m, thr))
                    def _():
                        for j in range(nl):
                            scan_chunk(s * 128 + j * 16)
                _tail()

            jn = jref[0]
            neg = jnp.full((16,), NEG, jnp.float32)
            mone = jnp.full((16,), -1, jnp.int32)

            def fold(t, T):
                T0v, T0i, T1v, T1i = T
                xv = cvref[pl.ds(t * 16, 16)]
                xi = ciref[pl.ds(t * 16, 16)]
                xvs, xis = _srt(xv, xi)
                return _merge16_into32(T0v, T0i, T1v, T1i, xvs, xis)

            T0v, T0i, T1v, T1i = lax.fori_loop(
                0, jn, fold, (neg, mone, neg, mone))

            s20 = jnp.sum(T0v) + jnp.sum(jnp.where(ii < 4, T1v, 0.0))
            den = jnp.full((16,), 1e-9, jnp.float32) + s20
            st_i0[pl.ds(0, 16)] = T0i
            st_i1[pl.ds(0, 16)] = T1i
            st_w0[pl.ds(0, 16)] = T0v / den
            st_w1[pl.ds(0, 16)] = T1v / den
            pltpu.sync_copy(st_i0, idx_out.at[row, pl.ds(0, 16)])
            pltpu.sync_copy(st_i1, idx_out.at[row, pl.ds(16, 16)])
            pltpu.sync_copy(st_w0, w_out.at[row, pl.ds(0, 16)])
            pltpu.sync_copy(st_w1, w_out.at[row, pl.ds(16, 16)])

        pltpu.async_copy(src(r0), bufA, semA)
        pltpu.async_copy(src(r0 + 1), bufB, semB)

        def rowpair(i, carry):
            rowA = r0 + 2 * i
            pltpu.make_async_copy(src(rowA), bufA, semA).wait()
            process(bufA, rowA)
            pltpu.async_copy(src(jnp.minimum(rowA + 2, B - 1)), bufA, semA)
            rowB = rowA + 1
            pltpu.make_async_copy(src(rowB), bufB, semB).wait()
            process(bufB, rowB)
            pltpu.async_copy(src(jnp.minimum(rowB + 2, B - 1)), bufB, semB)
            return carry

        lax.fori_loop(0, RPW // 2, rowpair, 0)
        pltpu.make_async_copy(src(r0), bufA, semA).wait()
        pltpu.make_async_copy(src(r0), bufB, semB).wait()

    return topk_kernel(tflat)


def _gather_call(memory, edge_features, node_last_update, idx2d):
    info = plsc.get_sparse_core_info()
    NW = info.num_cores * info.num_subcores
    NB = (B * K) // (NW * 128)
    mesh = plsc.VectorSubcoreMesh(core_axis_name="c", subcore_axis_name="s")

    @functools.partial(
        pl.kernel,
        out_type=(jax.ShapeDtypeStruct((B * K, D), jnp.float32),
                  jax.ShapeDtypeStruct((B * K, E_FEAT), jnp.float32),
                  jax.ShapeDtypeStruct((B * K,), jnp.float32)),
        mesh=mesh,
        scratch_types=[
            pltpu.VMEM((NB, 128), jnp.int32),
            pltpu.VMEM((NB, 128), jnp.int32),
            pltpu.VMEM((NB * 128, D), jnp.float32),
            pltpu.VMEM((NB * 128, E_FEAT), jnp.float32),
            pltpu.VMEM((NB * 128,), jnp.float32),
            pltpu.SemaphoreType.DMA,
            pltpu.SemaphoreType.DMA,
            pltpu.SemaphoreType.DMA,
        ],
        compiler_params=pltpu.CompilerParams(needs_layout_passes=False,
                                             use_tc_tiling_on_sc=False),
    )
    def gather_kernel(mem_hbm, ef_hbm, nlu_hbm, idx_hbm,
                      ng_out, eg_out, lg_out,
                      ixv, exv, nbuf, ebuf, lbuf, semN, semE, semL):
        wid = lax.axis_index("s") * info.num_cores + lax.axis_index("c")
        base = wid * NB * 128
        pltpu.sync_copy(idx_hbm.at[wid], ixv)
        for r in range(NB):
            for c in range(8):
                e = ixv[r, pl.ds(c * 16, 16)] * 31
                exv[r, pl.ds(c * 16, 16)] = lax.rem(
                    e, jnp.full((16,), N_EDGES, jnp.int32))
        cps = []
        for b in range(NB):
            cps.append(pltpu.async_copy(
                mem_hbm.at[ixv.at[b]], nbuf.at[pl.ds(b * 128, 128)], semN))
            cps.append(pltpu.async_copy(
                ef_hbm.at[exv.at[b]], ebuf.at[pl.ds(b * 128, 128)], semE))
            cps.append(pltpu.async_copy(
                nlu_hbm.at[ixv.at[b]], lbuf.at[pl.ds(b * 128, 128)], semL))
        for cp in cps:
            cp.wait()
        pltpu.sync_copy(nbuf, ng_out.at[pl.ds(base, NB * 128)])
        pltpu.sync_copy(ebuf, eg_out.at[pl.ds(base, NB * 128)])
        pltpu.sync_copy(lbuf, lg_out.at[pl.ds(base, NB * 128)])

    return gather_kernel(memory, edge_features, node_last_update, idx2d)


BB = 64
R = BB * K


def _mlp_body(neigh_ref, dt_ref, ef_ref, w_ref, src_ref,
              W1n_ref, W1t_ref, W1e_ref, b1_ref, W2_ref, b2_ref,
              W1s_ref, b1s_ref, W2s_ref, b2s_ref,
              Wca_ref, Wcs_ref, bc_ref, tw_ref, tb_ref,
              out_ref):
    dt = dt_ref[...]
    tenc = jnp.cos(dt * tw_ref[...] + tb_ref[...])
    h = (jnp.dot(neigh_ref[...], W1n_ref[...], preferred_element_type=jnp.float32)
         + jnp.dot(tenc, W1t_ref[...], preferred_element_type=jnp.float32)
         + jnp.dot(ef_ref[...], W1e_ref[...], preferred_element_type=jnp.float32)
         + b1_ref[...])
    h = jnp.maximum(h, 0.0)
    hw = h * w_ref[...]
    row = lax.broadcasted_iota(jnp.int32, (BB, R), 0)
    col = lax.broadcasted_iota(jnp.int32, (BB, R), 1)
    S0 = (col // K == row).astype(jnp.float32)
    aggpre = jnp.dot(S0, hw, preferred_element_type=jnp.float32)
    wsum = jnp.dot(S0, w_ref[...], preferred_element_type=jnp.float32)
    agg = (jnp.dot(aggpre, W2_ref[...], preferred_element_type=jnp.float32)
           + wsum * b2_ref[...])
    s = jnp.maximum(
        jnp.dot(src_ref[...], W1s_ref[...], preferred_element_type=jnp.float32)
        + b1s_ref[...], 0.0)
    s = jnp.dot(s, W2s_ref[...], preferred_element_type=jnp.float32) + b2s_ref[...]
    out_ref[...] = (jnp.dot(agg, Wca_ref[...], preferred_element_type=jnp.float32)
                    + jnp.dot(s, Wcs_ref[...], preferred_element_type=jnp.float32)
                    + bc_ref[...])


def _mlp_call(neigh, dt, ef, wflat, src,
              W1n, W1t, W1e, b1, W2, b2, W1s, b1s, W2s, b2s, Wca, Wcs, bc,
              tw, tb):
    grid = (B // BB,)
    full = lambda shape: pl.BlockSpec(shape, lambda i: (0, 0))
    return pl.pallas_call(
        _mlp_body,
        grid=grid,
        in_specs=[
            pl.BlockSpec((R, D), lambda i: (i, 0)),
            pl.BlockSpec((R, 1), lambda i: (i, 0)),
            pl.BlockSpec((R, E_FEAT), lambda i: (i, 0)),
            pl.BlockSpec((R, 1), lambda i: (i, 0)),
            pl.BlockSpec((BB, D), lambda i: (i, 0)),
            full((D, D)), full((T_FEAT, D)), full((E_FEAT, D)), full((1, D)),
            full((D, D)), full((1, D)),
            full((D, D)), full((1, D)), full((D, D)), full((1, D)),
            full((D, D)), full((D, D)), full((1, D)),
            full((1, T_FEAT)), full((1, T_FEAT)),
        ],
        out_specs=pl.BlockSpec((BB, D), lambda i: (i, 0)),
        out_shape=jax.ShapeDtypeStruct((B, D), jnp.float32),
    )(neigh, dt, ef, wflat, src,
      W1n, W1t, W1e, b1, W2, b2, W1s, b1s, W2s, b2s, Wca, Wcs, bc, tw, tb)


def kernel(memory, tppr_scores, timestamps, node_last_update, edge_features,
           W1, b1, W2, b2, W1s, b1s, W2s, b2s, Wc, bc, time_w, time_b,
           source_nodes):
    idx32, w32 = _topk_call(tppr_scores)
    idx = idx32[:, :K]
    w = w32[:, :K]
    idx2d = idx.reshape(32, B * K // (32 * 128), 128)
    neigh, ef, nlu = _gather_call(memory, edge_features, node_last_update,
                                  idx2d)
    dt = timestamps[:, None] - nlu.reshape(B, K)
    src = jnp.take(memory, source_nodes.astype(jnp.int32), axis=0)
    out = _mlp_call(
        neigh, dt.reshape(-1, 1), ef, w.reshape(-1, 1), src,
        W1[:D], W1[D:D + T_FEAT], W1[D + T_FEAT:], b1.reshape(1, D),
        W2, b2.reshape(1, D),
        W1s, b1s.reshape(1, D), W2s, b2s.reshape(1, D),
        Wc[:D], Wc[D:], bc.reshape(1, D),
        time_w.reshape(1, T_FEAT), time_b.reshape(1, T_FEAT),
    )
    return out


# speedup vs baseline: 1.2382x; 1.2382x over previous
import functools

import jax
import jax.numpy as jnp
from jax import lax
from jax.experimental import pallas as pl
from jax.experimental.pallas import tpu as pltpu
from jax.experimental.pallas import tpu_sc as plsc

N_NODES = 50000
N_EDGES = 1600000
B = 1024
D = 128
T_FEAT = 100
E_FEAT = 16
K = 20

NEG = -3.0e38
NSC = 392
NMACRO = 48
CAP = 96


def _srt(v, i):
    return plsc.sort_key_val(v, i, descending=True)


def _rev(x):
    return lax.rev(x, (0,))


def _merge16_into32(T0v, T0i, T1v, T1i, xv, xi):
    xrv, xri = _rev(xv), _rev(xi)
    s1 = xrv > T0v
    hv = jnp.where(s1, xrv, T0v)
    hi = jnp.where(s1, xri, T0i)
    lv = jnp.where(s1, T0v, xrv)
    li = jnp.where(s1, T0i, xri)
    nT0v, nT0i = _srt(hv, hi)
    lvs, lis = _srt(lv, li)
    lrv, lri = _rev(lvs), _rev(lis)
    s2 = lrv > T1v
    h2v = jnp.where(s2, lrv, T1v)
    h2i = jnp.where(s2, lri, T1i)
    nT1v, nT1i = _srt(h2v, h2i)
    return nT0v, nT0i, nT1v, nT1i


def _m32(a, b, ii):
    rb = _rev(b)
    hi = jnp.maximum(a, rb)
    lo = jnp.minimum(a, rb)
    hi, _ = _srt(hi, ii)
    lo, _ = _srt(lo, ii)
    return hi, lo


def _m32keep(a0, a1, b0, b1, ii):
    c0 = jnp.maximum(a0, _rev(b1))
    c1 = jnp.maximum(a1, _rev(b0))
    t0 = jnp.maximum(c0, c1)
    t1 = jnp.minimum(c0, c1)
    t0, _ = _srt(t0, ii)
    t1, _ = _srt(t1, ii)
    return t0, t1


def _topk_call(tflat):
    info = plsc.get_sparse_core_info()
    NW = info.num_cores * info.num_subcores
    RPW = B // NW
    mesh = plsc.VectorSubcoreMesh(core_axis_name="c", subcore_axis_name="s")

    @functools.partial(
        pl.kernel,
        out_type=(jax.ShapeDtypeStruct((B, 32), jnp.int32),
                  jax.ShapeDtypeStruct((B, 32), jnp.float32)),
        mesh=mesh,
        scratch_types=[
            pltpu.VMEM((N_NODES,), jnp.float32),
            pltpu.VMEM((N_NODES,), jnp.float32),
            pltpu.VMEM((NSC * 16,), jnp.float32),
            pltpu.VMEM(((NMACRO + 1) * 16,), jnp.float32),
            pltpu.VMEM((CAP * 16,), jnp.float32),
            pltpu.VMEM((CAP * 16,), jnp.int32),
            pltpu.VMEM((16,), jnp.int32),
            pltpu.VMEM((16,), jnp.float32),
            pltpu.VMEM((16,), jnp.int32),
            pltpu.VMEM((16,), jnp.float32),
            pltpu.SMEM((8,), jnp.int32),
            pltpu.SemaphoreType.DMA,
            pltpu.SemaphoreType.DMA,
        ],
        compiler_params=pltpu.CompilerParams(needs_layout_passes=False),
    )
    def topk_kernel(tppr_hbm, idx_out, w_out, bufA, bufB, Mref, GMref,
                    cvref, ciref,
                    st_i0, st_w0, st_i1, st_w1, jref, semA, semB):
        wid = lax.axis_index("s") * info.num_cores + lax.axis_index("c")
        r0 = wid * RPW
        ii = lax.iota(jnp.int32, 16)

        def src(row):
            return tppr_hbm.at[row]

        def anyge(x, thr):
            cnt = plsc.all_reduce_population_count(x >= thr)
            return lax.squeeze(lax.slice(cnt, (0,), (1,)), (0,)) > 0

        def process(buf, row):
            def macro(i, accs):
                out = []
                ms = []
                for a in range(8):
                    s = i * 8 + a
                    off = s * 128
                    l = [buf[pl.ds(off + j * 16, 16)] for j in range(8)]
                    m = jnp.maximum(
                        jnp.maximum(jnp.maximum(l[0], l[1]),
                                    jnp.maximum(l[2], l[3])),
                        jnp.maximum(jnp.maximum(l[4], l[5]),
                                    jnp.maximum(l[6], l[7])))
                    Mref[pl.ds(s * 16, 16)] = m
                    ms.append(m)
                    out.append(jnp.maximum(accs[a], m))
                gm = jnp.maximum(
                    jnp.maximum(jnp.maximum(ms[0], ms[1]),
                                jnp.maximum(ms[2], ms[3])),
                    jnp.maximum(jnp.maximum(ms[4], ms[5]),
                                jnp.maximum(ms[6], ms[7])))
                GMref[pl.ds(i * 16, 16)] = gm
                return tuple(out)

            neg16 = jnp.full((16,), NEG, jnp.float32)
            accs = lax.fori_loop(0, NMACRO, macro, (neg16,) * 8)
            accs = list(accs)
            ms = []
            for a in range(7):
                s = 384 + a
                off = s * 128
                nl = 8 if a < 6 else 5
                l = [buf[pl.ds(off + j * 16, 16)] for j in range(nl)]
                m = l[0]
                for j in range(1, nl):
                    m = jnp.maximum(m, l[j])
                Mref[pl.ds(s * 16, 16)] = m
                ms.append(m)
                accs[a] = jnp.maximum(accs[a], m)
            Mref[pl.ds(391 * 16, 16)] = neg16
            gm = ms[0]
            for a in range(1, 7):
                gm = jnp.maximum(gm, ms[a])
            GMref[pl.ds(NMACRO * 16, 16)] = gm

            sa = [_srt(accs[a], ii)[0] for a in range(8)]
            p0 = _m32(sa[0], sa[1], ii)
            p1 = _m32(sa[2], sa[3], ii)
            p2 = _m32(sa[4], sa[5], ii)
            p3 = _m32(sa[6], sa[7], ii)
            q0 = _m32keep(*p0, *p1, ii)
            q1 = _m32keep(*p2, *p3, ii)
            r_ = _m32keep(*q0, *q1, ii)
            thr = jnp.max(jnp.where(ii == 3, r_[1], NEG))

            jref[0] = 0

            def scan_chunk(off):
                x = buf[pl.ds(off, 16)]

                @pl.when(anyge(x, thr))
                def _():
                    jc = jnp.minimum(jref[0], CAP - 1)
                    cvref[pl.ds(jc * 16, 16)] = x
                    ciref[pl.ds(jc * 16, 16)] = ii + off
                    jref[0] = jc + 1

            def scan_grp(g, carry):
                gmv = GMref[pl.ds(g * 16, 16)]

                @pl.when(anyge(gmv, thr))
                def _():
                    for a in range(8):
                        s = g * 8 + a
                        m = Mref[pl.ds(s * 16, 16)]

                        @pl.when(anyge(m, thr))
                        def _():
                            for j in range(8):
                                scan_chunk(s * 128 + j * 16)
                return carry

            lax.fori_loop(0, NMACRO, scan_grp, 0)
            for a in range(7):
                s = 384 + a
                nl = 8 if a < 6 else 5

                def _tail(s=s, nl=nl, m=ms[a]):
                    @pl.when(anyge(m, thr))
                    def _():
                        for j in range(nl):
                            scan_chunk(s * 128 + j * 16)
                _tail()

            jn = jref[0]
            neg = jnp.full((16,), NEG, jnp.float32)
            mone = jnp.full((16,), -1, jnp.int32)

            def fold(t, T):
                T0v, T0i, T1v, T1i = T
                xv = cvref[pl.ds(t * 16, 16)]
                xi = ciref[pl.ds(t * 16, 16)]
                xvs, xis = _srt(xv, xi)
                return _merge16_into32(T0v, T0i, T1v, T1i, xvs, xis)

            T0v, T0i, T1v, T1i = lax.fori_loop(
                0, jn, fold, (neg, mone, neg, mone))

            s20 = jnp.sum(T0v) + jnp.sum(jnp.where(ii < 4, T1v, 0.0))
            den = jnp.full((16,), 1e-9, jnp.float32) + s20
            st_i0[pl.ds(0, 16)] = T0i
            st_i1[pl.ds(0, 16)] = T1i
            st_w0[pl.ds(0, 16)] = T0v / den
            st_w1[pl.ds(0, 16)] = T1v / den
            pltpu.sync_copy(st_i0, idx_out.at[row, pl.ds(0, 16)])
            pltpu.sync_copy(st_i1, idx_out.at[row, pl.ds(16, 16)])
            pltpu.sync_copy(st_w0, w_out.at[row, pl.ds(0, 16)])
            pltpu.sync_copy(st_w1, w_out.at[row, pl.ds(16, 16)])

        pltpu.async_copy(src(r0), bufA, semA)
        pltpu.async_copy(src(r0 + 1), bufB, semB)

        def rowpair(i, carry):
            rowA = r0 + 2 * i
            pltpu.make_async_copy(src(rowA), bufA, semA).wait()
            process(bufA, rowA)
            pltpu.async_copy(src(jnp.minimum(rowA + 2, B - 1)), bufA, semA)
            rowB = rowA + 1
            pltpu.make_async_copy(src(rowB), bufB, semB).wait()
            process(bufB, rowB)
            pltpu.async_copy(src(jnp.minimum(rowB + 2, B - 1)), bufB, semB)
            return carry

        lax.fori_loop(0, RPW // 2, rowpair, 0)
        pltpu.make_async_copy(src(r0), bufA, semA).wait()
        pltpu.make_async_copy(src(r0), bufB, semB).wait()

    return topk_kernel(tflat)


def _gather_call(memory, node_last_update, idx2d):
    info = plsc.get_sparse_core_info()
    NW = info.num_cores * info.num_subcores
    NB = (B * K) // (NW * 128)
    mesh = plsc.VectorSubcoreMesh(core_axis_name="c", subcore_axis_name="s")

    @functools.partial(
        pl.kernel,
        out_type=(jax.ShapeDtypeStruct((B * K, D), jnp.float32),
                  jax.ShapeDtypeStruct((B * K,), jnp.float32)),
        mesh=mesh,
        scratch_types=[
            pltpu.VMEM((NB, 128), jnp.int32),
            pltpu.VMEM((NB * 128, D), jnp.float32),
            pltpu.VMEM((NB * 128,), jnp.float32),
            pltpu.SemaphoreType.DMA,
            pltpu.SemaphoreType.DMA,
        ],
        compiler_params=pltpu.CompilerParams(needs_layout_passes=False),
    )
    def gather_kernel(mem_hbm, nlu_hbm, idx_hbm,
                      ng_out, lg_out,
                      ixv, nbuf, lbuf, semN, semL):
        wid = lax.axis_index("s") * info.num_cores + lax.axis_index("c")
        base = wid * NB * 128
        pltpu.sync_copy(idx_hbm.at[wid], ixv)
        cps = []
        for b in range(NB):
            cps.append(pltpu.async_copy(
                mem_hbm.at[ixv.at[b]], nbuf.at[pl.ds(b * 128, 128)], semN))
            cps.append(pltpu.async_copy(
                nlu_hbm.at[ixv.at[b]], lbuf.at[pl.ds(b * 128, 128)], semL))
        for cp in cps:
            cp.wait()
        pltpu.sync_copy(nbuf, ng_out.at[pl.ds(base, NB * 128)])
        pltpu.sync_copy(lbuf, lg_out.at[pl.ds(base, NB * 128)])

    return gather_kernel(memory, node_last_update, idx2d)


BB = 64
R = BB * K


def _mlp_body(neigh_ref, dt_ref, ef_ref, w_ref, src_ref,
              W1n_ref, W1t_ref, W1e_ref, b1_ref, W2_ref, b2_ref,
              W1s_ref, b1s_ref, W2s_ref, b2s_ref,
              Wca_ref, Wcs_ref, bc_ref, tw_ref, tb_ref,
              out_ref):
    dt = dt_ref[...]
    tenc = jnp.cos(dt * tw_ref[...] + tb_ref[...])
    h = (jnp.dot(neigh_ref[...], W1n_ref[...], preferred_element_type=jnp.float32)
         + jnp.dot(tenc, W1t_ref[...], preferred_element_type=jnp.float32)
         + jnp.dot(ef_ref[...], W1e_ref[...], preferred_element_type=jnp.float32)
         + b1_ref[...])
    h = jnp.maximum(h, 0.0)
    hw = h * w_ref[...]
    row = lax.broadcasted_iota(jnp.int32, (BB, R), 0)
    col = lax.broadcasted_iota(jnp.int32, (BB, R), 1)
    S0 = (col // K == row).astype(jnp.float32)
    aggpre = jnp.dot(S0, hw, preferred_element_type=jnp.float32)
    wsum = jnp.dot(S0, w_ref[...], preferred_element_type=jnp.float32)
    agg = (jnp.dot(aggpre, W2_ref[...], preferred_element_type=jnp.float32)
           + wsum * b2_ref[...])
    s = jnp.maximum(
        jnp.dot(src_ref[...], W1s_ref[...], preferred_element_type=jnp.float32)
        + b1s_ref[...], 0.0)
    s = jnp.dot(s, W2s_ref[...], preferred_element_type=jnp.float32) + b2s_ref[...]
    out_ref[...] = (jnp.dot(agg, Wca_ref[...], preferred_element_type=jnp.float32)
                    + jnp.dot(s, Wcs_ref[...], preferred_element_type=jnp.float32)
                    + bc_ref[...])


def _mlp_call(neigh, dt, ef, wflat, src,
              W1n, W1t, W1e, b1, W2, b2, W1s, b1s, W2s, b2s, Wca, Wcs, bc,
              tw, tb):
    grid = (B // BB,)
    full = lambda shape: pl.BlockSpec(shape, lambda i: (0, 0))
    return pl.pallas_call(
        _mlp_body,
        grid=grid,
        in_specs=[
            pl.BlockSpec((R, D), lambda i: (i, 0)),
            pl.BlockSpec((R, 1), lambda i: (i, 0)),
            pl.BlockSpec((R, E_FEAT), lambda i: (i, 0)),
            pl.BlockSpec((R, 1), lambda i: (i, 0)),
            pl.BlockSpec((BB, D), lambda i: (i, 0)),
            full((D, D)), full((T_FEAT, D)), full((E_FEAT, D)), full((1, D)),
            full((D, D)), full((1, D)),
            full((D, D)), full((1, D)), full((D, D)), full((1, D)),
            full((D, D)), full((D, D)), full((1, D)),
            full((1, T_FEAT)), full((1, T_FEAT)),
        ],
        out_specs=pl.BlockSpec((BB, D), lambda i: (i, 0)),
        out_shape=jax.ShapeDtypeStruct((B, D), jnp.float32),
    )(neigh, dt, ef, wflat, src,
      W1n, W1t, W1e, b1, W2, b2, W1s, b1s, W2s, b2s, Wca, Wcs, bc, tw, tb)


def kernel(memory, tppr_scores, timestamps, node_last_update, edge_features,
           W1, b1, W2, b2, W1s, b1s, W2s, b2s, Wc, bc, time_w, time_b,
           source_nodes):
    idx32, w32 = _topk_call(tppr_scores)
    idx = idx32[:, :K]
    w = w32[:, :K]
    idx2d = idx.reshape(32, B * K // (32 * 128), 128)
    neigh, nlu = _gather_call(memory, node_last_update, idx2d)
    dt = timestamps[:, None] - nlu.reshape(B, K)
    eid = (idx.reshape(-1) * 31) % N_EDGES
    ef = jnp.take(edge_features, eid, axis=0)
    src = jnp.take(memory, source_nodes.astype(jnp.int32), axis=0)
    out = _mlp_call(
        neigh, dt.reshape(-1, 1), ef, w.reshape(-1, 1), src,
        W1[:D], W1[D:D + T_FEAT], W1[D + T_FEAT:], b1.reshape(1, D),
        W2, b2.reshape(1, D),
        W1s, b1s.reshape(1, D), W2s, b2s.reshape(1, D),
        Wc[:D], Wc[D:], bc.reshape(1, D),
        time_w.reshape(1, T_FEAT), time_b.reshape(1, T_FEAT),
    )
    return out
